# baseline (speedup 1.0000x reference)
import functools

import jax
import jax.numpy as jnp
from jax import lax
from jax.experimental import pallas as pl
from jax.experimental.pallas import tpu as pltpu
from jax.experimental.pallas import tpu_sc as plsc

NC = 2
NS = 16


def _leaky(x, slope):
    return jnp.where(x >= 0, x, slope * x)


def _proj_body(te_ref, x_ref, wev_ref, bev_ref, gev_ref, beev_ref,
               wob_ref, bob_ref, gob_ref, beob_ref, wth_ref, bth_ref, o_ref):
    i = pl.program_id(0)
    blk = x_ref.shape[0]
    x = x_ref[...]
    he = _leaky(jnp.dot(x, wev_ref[...], preferred_element_type=jnp.float32)
                + bev_ref[...], 0.2)
    ho = _leaky(jnp.dot(x, wob_ref[...], preferred_element_type=jnp.float32)
                + bob_ref[...], 0.2)

    def ln(v, g, b):
        m = jnp.mean(v, axis=-1, keepdims=True)
        var = jnp.mean((v - m) ** 2, axis=-1, keepdims=True)
        return (v - m) / jnp.sqrt(var + 1e-5) * g + b

    he = ln(he, gev_ref[...], beev_ref[...])
    ho = ln(ho, gob_ref[...], beob_ref[...])
    row = i * blk + lax.broadcasted_iota(jnp.int32, (blk, 1), 0)
    sel = jnp.where(row < te_ref[0, 0], he, ho)
    o_ref[...] = (jnp.dot(sel, wth_ref[...], preferred_element_type=jnp.float32)
                  + bth_ref[...])


def _tc_proj(X, te, W_ev, b_ev, g_ev, be_ev, W_ob, b_ob, g_ob, be_ob, W_th, b_th):
    n, d = X.shape
    blk = 1000
    full = pl.BlockSpec((d, d), lambda i: (0, 0))
    vec = pl.BlockSpec((1, d), lambda i: (0, 0))
    return pl.pallas_call(
        _proj_body,
        grid=(n // blk,),
        in_specs=[
            pl.BlockSpec(memory_space=pltpu.SMEM),
            pl.BlockSpec((blk, d), lambda i: (i, 0)),
            full, vec, vec, vec,
            full, vec, vec, vec,
            full, vec,
        ],
        out_specs=pl.BlockSpec((blk, d), lambda i: (i, 0)),
        out_shape=jax.ShapeDtypeStruct((n, d), jnp.float32),
    )(te, X, W_ev, b_ev.reshape(1, d), g_ev.reshape(1, d), be_ev.reshape(1, d),
      W_ob, b_ob.reshape(1, d), g_ob.reshape(1, d), be_ob.reshape(1, d),
      W_th, b_th.reshape(1, d))


def _sc_pass(table, svals, idx_g, idx_s, nseg_pad, zrows, zvec, const_vals,
             K):
    nnz = idx_g.shape[0]
    d = table.shape[1]
    nchunks = nnz // K
    rps = nseg_pad // NS
    mesh = plsc.VectorSubcoreMesh(core_axis_name="c", subcore_axis_name="s")

    @functools.partial(
        pl.kernel,
        out_type=(jax.ShapeDtypeStruct((NC, nseg_pad, d), jnp.float32),
                  jax.ShapeDtypeStruct((NC, 8, nseg_pad), jnp.float32)),
        mesh=mesh,
        scratch_types=[
            [pltpu.VMEM((K,), jnp.int32)] * 2,
            [pltpu.VMEM((K,), jnp.int32)] * 2,
            [pltpu.VMEM((K, d), jnp.float32)] * 2,
            [pltpu.VMEM((K,), jnp.float32)] * 2,
            pltpu.VMEM_SHARED(((K if const_vals else svals.shape[0]),),
                              jnp.float32),
            pltpu.VMEM_SHARED((nseg_pad, d), jnp.float32),
            pltpu.VMEM_SHARED((nseg_pad,), jnp.float32),
            [pltpu.SemaphoreType.DMA] * 2,
            [pltpu.SemaphoreType.DMA] * 2,
            [pltpu.SemaphoreType.DMA] * 2,
            [pltpu.SemaphoreType.DMA] * 2,
            [pltpu.SemaphoreType.DMA] * 2,
        ],
    )
    def k(table_h, sval_h, idxg_h, idxs_h, zrows_h, zvec_h,
          out_rows, out_v, idxg_v, idxs_v, rows_v, vals_v, sv_sh, acc, accv,
          semi, semr, semv, semsr, semsv):
        c = lax.axis_index("c")
        s = lax.axis_index("s")
        r0 = s * rps
        pltpu.sync_copy(zrows_h.at[pl.ds(r0, rps)], acc.at[pl.ds(r0, rps)])

        @pl.when(s == 0)
        def _():
            pltpu.sync_copy(zvec_h.at[pl.ds(0, nseg_pad)], accv)
        if const_vals:
            pltpu.sync_copy(sval_h.at[pl.ds(0, K)], vals_v[0])
            pltpu.sync_copy(sval_h.at[pl.ds(0, K)], vals_v[1])
        else:

            @pl.when(s == 0)
            def _():
                pltpu.sync_copy(sval_h, sv_sh)

        wid = c * NS + s
        ntile = (nchunks - wid + NC * NS - 1) // (NC * NS)

        def start_idx(b, i):
            base = (wid + (NC * NS) * i) * K
            pltpu.async_copy(idxg_h.at[pl.ds(base, K)], idxg_v[b], semi[b])
            pltpu.async_copy(idxs_h.at[pl.ds(base, K)], idxs_v[b], semi[b])

        def wait_idx(b):
            pltpu.make_async_copy(idxg_h.at[pl.ds(0, K)], idxg_v[b],
                                  semi[b]).wait()
            pltpu.make_async_copy(idxs_h.at[pl.ds(0, K)], idxs_v[b],
                                  semi[b]).wait()

        def start_gather(b):
            pltpu.async_copy(table_h.at[idxg_v[b]], rows_v[b], semr[b])
            if not const_vals:
                pltpu.async_copy(sv_sh.at[idxg_v[b]], vals_v[b], semv[b])

        def wait_gather(b):
            pltpu.make_async_copy(table_h.at[idxg_v[b]], rows_v[b],
                                  semr[b]).wait()
            if not const_vals:
                pltpu.make_async_copy(sv_sh.at[idxg_v[b]], vals_v[b],
                                      semv[b]).wait()

        def scatter(b):
            r = pltpu.async_copy(rows_v[b], acc.at[idxs_v[b]], semsr[b],
                                 add=True)
            v = pltpu.async_copy(vals_v[b], accv.at[idxs_v[b]], semsv[b],
                                 add=True)
            r.wait()
            v.wait()

        start_idx(0, 0)
        start_idx(1, 1)
        plsc.subcore_barrier()
        wait_idx(0)
        start_gather(0)

        def body(i, carry):
            for b in (0, 1):
                nb = 1 - b

                @pl.when(lax.rem(i, 2) == b)
                def _():
                    @pl.when(i + 1 < ntile)
                    def _():
                        wait_idx(nb)
                        start_gather(nb)

                    wait_gather(b)
                    scatter(b)

                    @pl.when(i + 2 < ntile)
                    def _():
                        start_idx(b, i + 2)

            return carry

        lax.fori_loop(0, ntile, body, 0)
        plsc.subcore_barrier()
        pltpu.sync_copy(acc.at[pl.ds(r0, rps)], out_rows.at[c, pl.ds(r0, rps)])

        @pl.when(s == 0)
        def _():
            pltpu.sync_copy(accv, out_v.at[c, 0])

    return k(table, svals, idx_g, idx_s, zrows, zvec)


def _mid_body(sp_ref, cp_ref, ae_ref, ytil_ref, exe_ref):
    p = sp_ref[0] + sp_ref[1]
    c = cp_ref[0, 0] + cp_ref[1, 0]
    y = p / jnp.maximum(c, 1.0)[:, None]
    alpha = jnp.dot(y, ae_ref[...], preferred_element_type=jnp.float32)
    sc = _leaky(alpha, 0.2) * 5.0
    g = jnp.max(sc)
    exe = jnp.exp(sc - g)
    exe_ref[...] = exe
    ytil_ref[...] = y * exe


def _tc_mid(sp, cp, a_e):
    ncc, ep, d = sp.shape
    return pl.pallas_call(
        _mid_body,
        out_shape=(jax.ShapeDtypeStruct((ep, d), jnp.float32),
                   jax.ShapeDtypeStruct((ep, 1), jnp.float32)),
    )(sp, cp, a_e.reshape(d, 1))


def _out_body(zp_ref, dp_ref, o_ref):
    z = zp_ref[0] + zp_ref[1]
    den = (dp_ref[0, 0] + dp_ref[1, 0])[:, None]
    pos = den > 0
    xo = jnp.where(pos, z / jnp.where(pos, den, 1.0), 0.0)
    o_ref[...] = jnp.where(xo > 0, xo, jnp.exp(jnp.minimum(xo, 0.0)) - 1.0)


def _tc_out(zp, dp):
    ncc, npad, d = zp.shape
    return pl.pallas_call(
        _out_body,
        out_shape=jax.ShapeDtypeStruct((npad, d), jnp.float32),
    )(zp, dp)


def kernel(X, pair_v, pair_e, total_events, W_ev, b_ev, g_ev, be_ev,
           W_ob, b_ob, g_ob, be_ob, W_th, b_th, a_e):
    n, d = X.shape
    ep = 5120
    npad = 10112

    te = jnp.asarray(total_events, jnp.int32).reshape(1, 1)
    X1 = _tc_proj(X, te, W_ev, b_ev, g_ev, be_ev, W_ob, b_ob, g_ob, be_ob,
                  W_th, b_th)

    zrows = jnp.zeros((npad, d), jnp.float32)
    zvec = jnp.zeros((npad,), jnp.float32)
    ones = jnp.ones((n,), jnp.float32)

    sp, cp = _sc_pass(X1, ones, pair_v, pair_e, ep, zrows, zvec, True, 320)
    ytil, exe = _tc_mid(sp, cp, a_e)
    kb = 184
    nnz = pair_v.shape[0]
    pad = (-nnz) % kb
    pe_b = jnp.concatenate([pair_e, jnp.zeros((pad,), jnp.int32)])
    pv_b = jnp.concatenate([pair_v, jnp.full((pad,), npad - 1, jnp.int32)])
    zp, dp = _sc_pass(ytil, exe.reshape(ep), pe_b, pv_b, npad, zrows, zvec,
                      False, kb)
    out = _tc_out(zp, dp)
    return out[:n]


# speedup vs baseline: 1.0271x; 1.0271x over previous
import functools

import jax
import jax.numpy as jnp
from jax import lax
from jax.experimental import pallas as pl
from jax.experimental.pallas import tpu as pltpu
from jax.experimental.pallas import tpu_sc as plsc

NC = 2
NS = 16


def _leaky(x, slope):
    return jnp.where(x >= 0, x, slope * x)


def _proj_body(te_ref, x_ref, wev_ref, bev_ref, gev_ref, beev_ref,
               wob_ref, bob_ref, gob_ref, beob_ref, wth_ref, bth_ref, o_ref):
    i = pl.program_id(0)
    blk = x_ref.shape[0]
    x = x_ref[...]
    he = _leaky(jnp.dot(x, wev_ref[...], preferred_element_type=jnp.float32)
                + bev_ref[...], 0.2)
    ho = _leaky(jnp.dot(x, wob_ref[...], preferred_element_type=jnp.float32)
                + bob_ref[...], 0.2)

    def ln(v, g, b):
        m = jnp.mean(v, axis=-1, keepdims=True)
        var = jnp.mean((v - m) ** 2, axis=-1, keepdims=True)
        return (v - m) / jnp.sqrt(var + 1e-5) * g + b

    he = ln(he, gev_ref[...], beev_ref[...])
    ho = ln(ho, gob_ref[...], beob_ref[...])
    row = i * blk + lax.broadcasted_iota(jnp.int32, (blk, 1), 0)
    sel = jnp.where(row < te_ref[0, 0], he, ho)
    o_ref[...] = (jnp.dot(sel, wth_ref[...], preferred_element_type=jnp.float32)
                  + bth_ref[...])


def _tc_proj(X, te, W_ev, b_ev, g_ev, be_ev, W_ob, b_ob, g_ob, be_ob, W_th, b_th):
    n, d = X.shape
    blk = 1000
    full = pl.BlockSpec((d, d), lambda i: (0, 0))
    vec = pl.BlockSpec((1, d), lambda i: (0, 0))
    return pl.pallas_call(
        _proj_body,
        grid=(n // blk,),
        in_specs=[
            pl.BlockSpec(memory_space=pltpu.SMEM),
            pl.BlockSpec((blk, d), lambda i: (i, 0)),
            full, vec, vec, vec,
            full, vec, vec, vec,
            full, vec,
        ],
        out_specs=pl.BlockSpec((blk, d), lambda i: (i, 0)),
        out_shape=jax.ShapeDtypeStruct((n, d), jnp.float32),
    )(te, X, W_ev, b_ev.reshape(1, d), g_ev.reshape(1, d), be_ev.reshape(1, d),
      W_ob, b_ob.reshape(1, d), g_ob.reshape(1, d), be_ob.reshape(1, d),
      W_th, b_th.reshape(1, d))


def _sc_pass(table, svals, idx_g, idx_s, nseg_pad, zrows, zvec, const_vals,
             K):
    nnz = idx_g.shape[0]
    d = table.shape[1]
    nchunks = nnz // K
    rps = nseg_pad // NS
    mesh = plsc.VectorSubcoreMesh(core_axis_name="c", subcore_axis_name="s")

    @functools.partial(
        pl.kernel,
        out_type=(jax.ShapeDtypeStruct((NC, nseg_pad, d), jnp.float32),
                  jax.ShapeDtypeStruct((NC, 8, nseg_pad), jnp.float32)),
        mesh=mesh,
        scratch_types=[
            [pltpu.VMEM((K,), jnp.int32)] * 2,
            [pltpu.VMEM((K,), jnp.int32)] * 2,
            [pltpu.VMEM((K, d), jnp.float32)] * 2,
            [pltpu.VMEM((K,), jnp.float32)] * 2,
            pltpu.VMEM_SHARED(((K if const_vals else svals.shape[0]),),
                              jnp.float32),
            pltpu.VMEM_SHARED((nseg_pad, d), jnp.float32),
            pltpu.VMEM_SHARED((nseg_pad,), jnp.float32),
            [pltpu.SemaphoreType.DMA] * 2,
            [pltpu.SemaphoreType.DMA] * 2,
            [pltpu.SemaphoreType.DMA] * 2,
            [pltpu.SemaphoreType.DMA] * 2,
            [pltpu.SemaphoreType.DMA] * 2,
        ],
    )
    def k(table_h, sval_h, idxg_h, idxs_h, zrows_h, zvec_h,
          out_rows, out_v, idxg_v, idxs_v, rows_v, vals_v, sv_sh, acc, accv,
          semi, semr, semv, semsr, semsv):
        c = lax.axis_index("c")
        s = lax.axis_index("s")
        r0 = s * rps
        pltpu.sync_copy(zrows_h.at[pl.ds(r0, rps)], acc.at[pl.ds(r0, rps)])

        @pl.when(s == 0)
        def _():
            pltpu.sync_copy(zvec_h.at[pl.ds(0, nseg_pad)], accv)
        if const_vals:
            pltpu.sync_copy(sval_h.at[pl.ds(0, K)], vals_v[0])
            pltpu.sync_copy(sval_h.at[pl.ds(0, K)], vals_v[1])
        else:

            @pl.when(s == 0)
            def _():
                pltpu.sync_copy(sval_h, sv_sh)

        wid = c * NS + s
        ntile = (nchunks - wid + NC * NS - 1) // (NC * NS)

        def start_idx(b, i):
            base = (wid + (NC * NS) * i) * K
            pltpu.async_copy(idxg_h.at[pl.ds(base, K)], idxg_v[b], semi[b])
            pltpu.async_copy(idxs_h.at[pl.ds(base, K)], idxs_v[b], semi[b])

        def wait_idx(b):
            pltpu.make_async_copy(idxg_h.at[pl.ds(0, K)], idxg_v[b],
                                  semi[b]).wait()
            pltpu.make_async_copy(idxs_h.at[pl.ds(0, K)], idxs_v[b],
                                  semi[b]).wait()

        def start_gather(b):
            pltpu.async_copy(table_h.at[idxg_v[b]], rows_v[b], semr[b])
            if not const_vals:
                pltpu.async_copy(sv_sh.at[idxg_v[b]], vals_v[b], semv[b])

        def wait_gather(b):
            pltpu.make_async_copy(table_h.at[idxg_v[b]], rows_v[b],
                                  semr[b]).wait()
            if not const_vals:
                pltpu.make_async_copy(sv_sh.at[idxg_v[b]], vals_v[b],
                                      semv[b]).wait()

        def scatter(b):
            r = pltpu.async_copy(rows_v[b], acc.at[idxs_v[b]], semsr[b],
                                 add=True)
            v = pltpu.async_copy(vals_v[b], accv.at[idxs_v[b]], semsv[b],
                                 add=True)
            r.wait()
            v.wait()

        start_idx(0, 0)
        start_idx(1, 1)
        plsc.subcore_barrier()
        wait_idx(0)
        start_gather(0)

        def body(i, carry):
            for b in (0, 1):
                nb = 1 - b

                @pl.when(lax.rem(i, 2) == b)
                def _():
                    @pl.when(i + 1 < ntile)
                    def _():
                        wait_idx(nb)
                        start_gather(nb)

                    wait_gather(b)
                    scatter(b)

                    @pl.when(i + 2 < ntile)
                    def _():
                        start_idx(b, i + 2)

            return carry

        lax.fori_loop(0, ntile, body, 0)
        plsc.subcore_barrier()
        pltpu.sync_copy(acc.at[pl.ds(r0, rps)], out_rows.at[c, pl.ds(r0, rps)])

        @pl.when(s == 0)
        def _():
            pltpu.sync_copy(accv, out_v.at[c, 0])

    return k(table, svals, idx_g, idx_s, zrows, zvec)


def _mid_body(sp_ref, cp_ref, ae_ref, ytil_ref, exe_ref):
    p = sp_ref[0] + sp_ref[1]
    c = cp_ref[0, 0] + cp_ref[1, 0]
    y = p / jnp.maximum(c, 1.0)[:, None]
    alpha = jnp.dot(y, ae_ref[...], preferred_element_type=jnp.float32)
    sc = _leaky(alpha, 0.2) * 5.0
    g = jnp.max(sc)
    exe = jnp.exp(sc - g)
    exe_ref[...] = exe
    ytil_ref[...] = y * exe


def _tc_mid(sp, cp, a_e):
    ncc, ep, d = sp.shape
    return pl.pallas_call(
        _mid_body,
        out_shape=(jax.ShapeDtypeStruct((ep, d), jnp.float32),
                   jax.ShapeDtypeStruct((ep, 1), jnp.float32)),
    )(sp, cp, a_e.reshape(d, 1))


def _out_body(zp_ref, dp_ref, o_ref):
    z = zp_ref[0] + zp_ref[1]
    den = (dp_ref[0, 0] + dp_ref[1, 0])[:, None]
    pos = den > 0
    xo = jnp.where(pos, z / jnp.where(pos, den, 1.0), 0.0)
    o_ref[...] = jnp.where(xo > 0, xo, jnp.exp(jnp.minimum(xo, 0.0)) - 1.0)


def _tc_out(zp, dp):
    ncc, npad, d = zp.shape
    return pl.pallas_call(
        _out_body,
        out_shape=jax.ShapeDtypeStruct((npad, d), jnp.float32),
    )(zp, dp)


def kernel(X, pair_v, pair_e, total_events, W_ev, b_ev, g_ev, be_ev,
           W_ob, b_ob, g_ob, be_ob, W_th, b_th, a_e):
    n, d = X.shape
    ep = 5120
    npad = 10112

    te = jnp.asarray(total_events, jnp.int32).reshape(1, 1)
    X1 = _tc_proj(X, te, W_ev, b_ev, g_ev, be_ev, W_ob, b_ob, g_ob, be_ob,
                  W_th, b_th)

    zrows = jnp.zeros((npad, d), jnp.float32)
    zvec = jnp.zeros((npad,), jnp.float32)
    ones = jnp.ones((n,), jnp.float32)

    sp, cp = _sc_pass(X1, ones, pair_v, pair_e, ep, zrows, zvec, True, 320)
    ytil, exe = _tc_mid(sp, cp, a_e)
    zp, dp = _sc_pass(ytil, exe.reshape(ep), pair_e, pair_v, npad, zrows, zvec,
                      False, 160)
    out = _tc_out(zp, dp)
    return out[:n]


# speedup vs baseline: 1.0387x; 1.0113x over previous
import functools

import jax
import jax.numpy as jnp
from jax import lax
from jax.experimental import pallas as pl
from jax.experimental.pallas import tpu as pltpu
from jax.experimental.pallas import tpu_sc as plsc

NC = 2
NS = 16


def _leaky(x, slope):
    return jnp.where(x >= 0, x, slope * x)


def _proj_body(te_ref, x_ref, wev_ref, bev_ref, gev_ref, beev_ref,
               wob_ref, bob_ref, gob_ref, beob_ref, wth_ref, bth_ref, o_ref):
    i = pl.program_id(0)
    blk = x_ref.shape[0]
    x = x_ref[...]
    he = _leaky(jnp.dot(x, wev_ref[...], preferred_element_type=jnp.float32)
                + bev_ref[...], 0.2)
    ho = _leaky(jnp.dot(x, wob_ref[...], preferred_element_type=jnp.float32)
                + bob_ref[...], 0.2)

    def ln(v, g, b):
        m = jnp.mean(v, axis=-1, keepdims=True)
        var = jnp.mean((v - m) ** 2, axis=-1, keepdims=True)
        return (v - m) / jnp.sqrt(var + 1e-5) * g + b

    he = ln(he, gev_ref[...], beev_ref[...])
    ho = ln(ho, gob_ref[...], beob_ref[...])
    row = i * blk + lax.broadcasted_iota(jnp.int32, (blk, 1), 0)
    sel = jnp.where(row < te_ref[0, 0], he, ho)
    o_ref[...] = (jnp.dot(sel, wth_ref[...], preferred_element_type=jnp.float32)
                  + bth_ref[...])


def _tc_proj(X, te, W_ev, b_ev, g_ev, be_ev, W_ob, b_ob, g_ob, be_ob, W_th, b_th):
    n, d = X.shape
    blk = 1000
    full = pl.BlockSpec((d, d), lambda i: (0, 0))
    vec = pl.BlockSpec((1, d), lambda i: (0, 0))
    return pl.pallas_call(
        _proj_body,
        grid=(n // blk,),
        in_specs=[
            pl.BlockSpec(memory_space=pltpu.SMEM),
            pl.BlockSpec((blk, d), lambda i: (i, 0)),
            full, vec, vec, vec,
            full, vec, vec, vec,
            full, vec,
        ],
        out_specs=pl.BlockSpec((blk, d), lambda i: (i, 0)),
        out_shape=jax.ShapeDtypeStruct((n, d), jnp.float32),
    )(te, X, W_ev, b_ev.reshape(1, d), g_ev.reshape(1, d), be_ev.reshape(1, d),
      W_ob, b_ob.reshape(1, d), g_ob.reshape(1, d), be_ob.reshape(1, d),
      W_th, b_th.reshape(1, d))


def _sc_pass(table, svals, idx_g, idx_s, nseg_pad, zrows, zvec, const_vals,
             K):
    nnz = idx_g.shape[0]
    d = table.shape[1]
    nchunks = nnz // K
    rps = nseg_pad // NS
    mesh = plsc.VectorSubcoreMesh(core_axis_name="c", subcore_axis_name="s")

    @functools.partial(
        pl.kernel,
        out_type=(jax.ShapeDtypeStruct((NC, nseg_pad, d), jnp.float32),
                  jax.ShapeDtypeStruct((NC, 8, nseg_pad), jnp.float32)),
        mesh=mesh,
        scratch_types=[
            [pltpu.VMEM((K,), jnp.int32)] * 2,
            [pltpu.VMEM((K,), jnp.int32)] * 2,
            [pltpu.VMEM((K, d), jnp.float32)] * 2,
            [pltpu.VMEM((K,), jnp.float32)] * 2,
            pltpu.VMEM_SHARED(((K if const_vals else svals.shape[0]),),
                              jnp.float32),
            pltpu.VMEM_SHARED((nseg_pad, d), jnp.float32),
            pltpu.VMEM_SHARED((nseg_pad,), jnp.float32),
            [pltpu.SemaphoreType.DMA] * 2,
            [pltpu.SemaphoreType.DMA] * 2,
            [pltpu.SemaphoreType.DMA] * 2,
            [pltpu.SemaphoreType.DMA] * 2,
            [pltpu.SemaphoreType.DMA] * 2,
            pltpu.SemaphoreType.DMA,
        ],
    )
    def k(table_h, sval_h, idxg_h, idxs_h, zrows_h, zvec_h,
          out_rows, out_v, idxg_v, idxs_v, rows_v, vals_v, sv_sh, acc, accv,
          semi, semr, semv, semsr, semsv, semz):
        c = lax.axis_index("c")
        s = lax.axis_index("s")
        r0 = s * rps
        z = pltpu.async_copy(zrows_h.at[pl.ds(r0, rps)], acc.at[pl.ds(r0, rps)],
                             semz)

        @pl.when(s == 0)
        def _():
            pltpu.sync_copy(zvec_h.at[pl.ds(0, nseg_pad)], accv)
        if const_vals:
            pltpu.sync_copy(sval_h.at[pl.ds(0, K)], vals_v[0])
            pltpu.sync_copy(sval_h.at[pl.ds(0, K)], vals_v[1])
        else:

            @pl.when(s == 0)
            def _():
                pltpu.sync_copy(sval_h, sv_sh)

        wid = c * NS + s
        ntile = (nchunks - wid + NC * NS - 1) // (NC * NS)

        def start_idx(b, i):
            base = (wid + (NC * NS) * i) * K
            pltpu.async_copy(idxg_h.at[pl.ds(base, K)], idxg_v[b], semi[b])
            pltpu.async_copy(idxs_h.at[pl.ds(base, K)], idxs_v[b], semi[b])

        def wait_idx(b):
            pltpu.make_async_copy(idxg_h.at[pl.ds(0, K)], idxg_v[b],
                                  semi[b]).wait()
            pltpu.make_async_copy(idxs_h.at[pl.ds(0, K)], idxs_v[b],
                                  semi[b]).wait()

        def start_gather(b):
            pltpu.async_copy(table_h.at[idxg_v[b]], rows_v[b], semr[b])
            if not const_vals:
                pltpu.async_copy(sv_sh.at[idxg_v[b]], vals_v[b], semv[b])

        def wait_gather(b):
            pltpu.make_async_copy(table_h.at[idxg_v[b]], rows_v[b],
                                  semr[b]).wait()
            if not const_vals:
                pltpu.make_async_copy(sv_sh.at[idxg_v[b]], vals_v[b],
                                      semv[b]).wait()

        def scatter(b):
            r = pltpu.async_copy(rows_v[b], acc.at[idxs_v[b]], semsr[b],
                                 add=True)
            v = pltpu.async_copy(vals_v[b], accv.at[idxs_v[b]], semsv[b],
                                 add=True)
            r.wait()
            v.wait()

        start_idx(0, 0)
        start_idx(1, 1)
        wait_idx(0)
        pltpu.async_copy(table_h.at[idxg_v[0]], rows_v[0], semr[0])
        z.wait()
        plsc.subcore_barrier()
        if not const_vals:
            pltpu.async_copy(sv_sh.at[idxg_v[0]], vals_v[0], semv[0])

        def body(i, carry):
            for b in (0, 1):
                nb = 1 - b

                @pl.when(lax.rem(i, 2) == b)
                def _():
                    @pl.when(i + 1 < ntile)
                    def _():
                        wait_idx(nb)
                        start_gather(nb)

                    wait_gather(b)
                    scatter(b)

                    @pl.when(i + 2 < ntile)
                    def _():
                        start_idx(b, i + 2)

            return carry

        lax.fori_loop(0, ntile, body, 0)
        plsc.subcore_barrier()
        pltpu.sync_copy(acc.at[pl.ds(r0, rps)], out_rows.at[c, pl.ds(r0, rps)])

        @pl.when(s == 0)
        def _():
            pltpu.sync_copy(accv, out_v.at[c, 0])

    return k(table, svals, idx_g, idx_s, zrows, zvec)


def _mid_body(sp_ref, cp_ref, ae_ref, ytil_ref, exe_ref):
    p = sp_ref[0] + sp_ref[1]
    c = cp_ref[0, 0] + cp_ref[1, 0]
    y = p / jnp.maximum(c, 1.0)[:, None]
    alpha = jnp.dot(y, ae_ref[...], preferred_element_type=jnp.float32)
    sc = _leaky(alpha, 0.2) * 5.0
    g = jnp.max(sc)
    exe = jnp.exp(sc - g)
    exe_ref[...] = exe
    ytil_ref[...] = y * exe


def _tc_mid(sp, cp, a_e):
    ncc, ep, d = sp.shape
    return pl.pallas_call(
        _mid_body,
        out_shape=(jax.ShapeDtypeStruct((ep, d), jnp.float32),
                   jax.ShapeDtypeStruct((ep, 1), jnp.float32)),
    )(sp, cp, a_e.reshape(d, 1))


def _out_body(zp_ref, dp_ref, o_ref):
    z = zp_ref[0] + zp_ref[1]
    den = (dp_ref[0, 0] + dp_ref[1, 0])[:, None]
    pos = den > 0
    xo = jnp.where(pos, z / jnp.where(pos, den, 1.0), 0.0)
    o_ref[...] = jnp.where(xo > 0, xo, jnp.exp(jnp.minimum(xo, 0.0)) - 1.0)


def _tc_out(zp, dp):
    ncc, npad, d = zp.shape
    return pl.pallas_call(
        _out_body,
        out_shape=jax.ShapeDtypeStruct((npad, d), jnp.float32),
    )(zp, dp)


def kernel(X, pair_v, pair_e, total_events, W_ev, b_ev, g_ev, be_ev,
           W_ob, b_ob, g_ob, be_ob, W_th, b_th, a_e):
    n, d = X.shape
    ep = 5120
    npad = 10112

    te = jnp.asarray(total_events, jnp.int32).reshape(1, 1)
    X1 = _tc_proj(X, te, W_ev, b_ev, g_ev, be_ev, W_ob, b_ob, g_ob, be_ob,
                  W_th, b_th)

    zrows = jnp.zeros((npad, d), jnp.float32)
    zvec = jnp.zeros((npad,), jnp.float32)
    ones = jnp.ones((n,), jnp.float32)

    sp, cp = _sc_pass(X1, ones, pair_v, pair_e, ep, zrows, zvec, True, 320)
    ytil, exe = _tc_mid(sp, cp, a_e)
    zp, dp = _sc_pass(ytil, exe.reshape(ep), pair_e, pair_v, npad, zrows, zvec,
                      False, 160)
    out = _tc_out(zp, dp)
    return out[:n]
